# baseline (speedup 1.0000x reference)
import jax
import jax.numpy as jnp
from jax.experimental import pallas as pl
from jax.experimental.pallas import tpu as pltpu

_EMBED = 2048
_HIDDEN = 8192
_BM = 512
_BH = 2048
_BC = 512

_INV_SQRT2 = 0.7071067811865476


def _mlp_body(x_ref, w1_ref, b1_ref, w2_ref, b2_ref, o_ref):
    j = pl.program_id(1)
    xb = x_ref[...]
    contrib = None
    for c in range(_BH // _BC):
        sl = pl.ds(c * _BC, _BC)
        h = jax.lax.dot_general(
            xb, w1_ref[:, sl],
            (((1,), (0,)), ((), ())),
            preferred_element_type=jnp.float32)
        h = h + b1_ref[:, sl]
        h = 0.5 * h * (1.0 + jax.lax.erf(h * _INV_SQRT2))
        part = jax.lax.dot_general(
            h.astype(jnp.bfloat16), w2_ref[sl, :],
            (((1,), (0,)), ((), ())),
            preferred_element_type=jnp.float32)
        contrib = part if contrib is None else contrib + part

    @pl.when(j == 0)
    def _init():
        o_ref[...] = contrib + b2_ref[...]

    @pl.when(j != 0)
    def _acc():
        o_ref[...] += contrib


def kernel(x, proj1, proj1_bias, proj2, proj2_bias, gate_w):
    del gate_w
    L, N, E = x.shape
    M = L * N
    xb = x.reshape(M, E).astype(jnp.bfloat16)
    w1t = proj1.T.astype(jnp.bfloat16)
    w2t = proj2.T.astype(jnp.bfloat16)
    b1 = proj1_bias.reshape(1, _HIDDEN)
    b2 = proj2_bias.reshape(1, _EMBED)

    grid = (M // _BM, _HIDDEN // _BH)
    out = pl.pallas_call(
        _mlp_body,
        grid=grid,
        in_specs=[
            pl.BlockSpec((_BM, _EMBED), lambda i, j: (i, 0)),
            pl.BlockSpec((_EMBED, _BH), lambda i, j: (0, j)),
            pl.BlockSpec((1, _BH), lambda i, j: (0, j)),
            pl.BlockSpec((_BH, _EMBED), lambda i, j: (j, 0)),
            pl.BlockSpec((1, _EMBED), lambda i, j: (0, 0)),
        ],
        out_specs=pl.BlockSpec((_BM, _EMBED), lambda i, j: (i, 0)),
        out_shape=jax.ShapeDtypeStruct((M, E), jnp.float32),
        compiler_params=pltpu.CompilerParams(
            dimension_semantics=("parallel", "arbitrary"),
        ),
    )(xb, w1t, b1, w2t, b2)
    return out.reshape(L, N, E)


# speedup vs baseline: 1.0622x; 1.0622x over previous
import jax
import jax.numpy as jnp
from jax.experimental import pallas as pl
from jax.experimental.pallas import tpu as pltpu

_EMBED = 2048
_HIDDEN = 8192
_BM = 512
_BH = 2048
_NCAST = 16

_INV_SQRT2 = 0.7071067811865476


def _cast_body(w1_ref, w2_ref, x_ref, w1o_ref, w2o_ref, xo_ref):
    w1o_ref[...] = w1_ref[...].astype(jnp.bfloat16)
    w2o_ref[...] = w2_ref[...].astype(jnp.bfloat16)
    xo_ref[...] = x_ref[...].astype(jnp.bfloat16)


def _mlp_body(x_ref, w1_ref, b1_ref, w2_ref, b2_ref, o_ref):
    j = pl.program_id(1)
    h = jax.lax.dot_general(
        x_ref[...], w1_ref[...],
        (((1,), (1,)), ((), ())),
        preferred_element_type=jnp.float32)
    h = h + b1_ref[...]
    h = 0.5 * h * (1.0 + jax.lax.erf(h * _INV_SQRT2))
    contrib = jax.lax.dot_general(
        h.astype(jnp.bfloat16), w2_ref[...],
        (((1,), (1,)), ((), ())),
        preferred_element_type=jnp.float32)

    @pl.when(j == 0)
    def _init():
        o_ref[...] = contrib + b2_ref[...]

    @pl.when(j != 0)
    def _acc():
        o_ref[...] += contrib


def kernel(x, proj1, proj1_bias, proj2, proj2_bias, gate_w):
    del gate_w
    L, N, E = x.shape
    M = L * N
    xf = x.reshape(M, E)

    cw1 = _HIDDEN // _NCAST
    cw2 = _EMBED // _NCAST
    cx = M // _NCAST
    w1b, w2b, xb = pl.pallas_call(
        _cast_body,
        grid=(_NCAST,),
        in_specs=[
            pl.BlockSpec((cw1, _EMBED), lambda i: (i, 0)),
            pl.BlockSpec((cw2, _HIDDEN), lambda i: (i, 0)),
            pl.BlockSpec((cx, _EMBED), lambda i: (i, 0)),
        ],
        out_specs=[
            pl.BlockSpec((cw1, _EMBED), lambda i: (i, 0)),
            pl.BlockSpec((cw2, _HIDDEN), lambda i: (i, 0)),
            pl.BlockSpec((cx, _EMBED), lambda i: (i, 0)),
        ],
        out_shape=[
            jax.ShapeDtypeStruct((_HIDDEN, _EMBED), jnp.bfloat16),
            jax.ShapeDtypeStruct((_EMBED, _HIDDEN), jnp.bfloat16),
            jax.ShapeDtypeStruct((M, E), jnp.bfloat16),
        ],
        compiler_params=pltpu.CompilerParams(
            dimension_semantics=("arbitrary",),
        ),
    )(proj1, proj2, xf)

    b1 = proj1_bias.reshape(1, _HIDDEN)
    b2 = proj2_bias.reshape(1, _EMBED)

    grid = (M // _BM, _HIDDEN // _BH)
    out = pl.pallas_call(
        _mlp_body,
        grid=grid,
        in_specs=[
            pl.BlockSpec((_BM, _EMBED), lambda i, j: (i, 0)),
            pl.BlockSpec((_BH, _EMBED), lambda i, j: (j, 0)),
            pl.BlockSpec((1, _BH), lambda i, j: (0, j)),
            pl.BlockSpec((_EMBED, _BH), lambda i, j: (0, j)),
            pl.BlockSpec((1, _EMBED), lambda i, j: (0, 0)),
        ],
        out_specs=pl.BlockSpec((_BM, _EMBED), lambda i, j: (i, 0)),
        out_shape=jax.ShapeDtypeStruct((M, E), jnp.float32),
        compiler_params=pltpu.CompilerParams(
            dimension_semantics=("parallel", "arbitrary"),
        ),
    )(xb, w1b, b1, w2b, b2)
    return out.reshape(L, N, E)


# speedup vs baseline: 1.1067x; 1.0418x over previous
import jax
import jax.numpy as jnp
from jax.experimental import pallas as pl
from jax.experimental.pallas import tpu as pltpu

_EMBED = 2048
_HIDDEN = 8192
_BM = 1024
_BH = 512

_INV_SQRT2 = 0.7071067811865476


def _mlp_body(x_ref, w1_ref, b1_ref, w2_ref, b2_ref, o_ref):
    j = pl.program_id(1)
    w1b = w1_ref[...].astype(jnp.bfloat16)
    h = jax.lax.dot_general(
        x_ref[...], w1b,
        (((1,), (1,)), ((), ())),
        preferred_element_type=jnp.float32)
    h = h + b1_ref[...]
    h = 0.5 * h * (1.0 + jax.lax.erf(h * _INV_SQRT2))
    w2b = w2_ref[...].astype(jnp.bfloat16)
    contrib = jax.lax.dot_general(
        h.astype(jnp.bfloat16), w2b,
        (((1,), (1,)), ((), ())),
        preferred_element_type=jnp.float32)

    @pl.when(j == 0)
    def _init():
        o_ref[...] = contrib + b2_ref[...]

    @pl.when(j != 0)
    def _acc():
        o_ref[...] += contrib


def kernel(x, proj1, proj1_bias, proj2, proj2_bias, gate_w):
    del gate_w
    L, N, E = x.shape
    M = L * N
    xb = x.reshape(M, E).astype(jnp.bfloat16)
    b1 = proj1_bias.reshape(1, _HIDDEN)
    b2 = proj2_bias.reshape(1, _EMBED)

    grid = (M // _BM, _HIDDEN // _BH)
    out = pl.pallas_call(
        _mlp_body,
        grid=grid,
        in_specs=[
            pl.BlockSpec((_BM, _EMBED), lambda i, j: (i, 0)),
            pl.BlockSpec((_BH, _EMBED), lambda i, j: (j, 0)),
            pl.BlockSpec((1, _BH), lambda i, j: (0, j)),
            pl.BlockSpec((_EMBED, _BH), lambda i, j: (0, j)),
            pl.BlockSpec((1, _EMBED), lambda i, j: (0, 0)),
        ],
        out_specs=pl.BlockSpec((_BM, _EMBED), lambda i, j: (i, 0)),
        out_shape=jax.ShapeDtypeStruct((M, E), jnp.float32),
        compiler_params=pltpu.CompilerParams(
            dimension_semantics=("parallel", "arbitrary"),
        ),
    )(xb, proj1, b1, proj2, b2)
    return out.reshape(L, N, E)
